# Initial kernel scaffold; baseline (speedup 1.0000x reference)
import jax
import jax.numpy as jnp
from jax.experimental import pallas as pl


def kernel(x_router_input, W, b, expert_biases):
    raise NotImplementedError("write your pallas kernel here")


# speedup vs baseline: 1.4148x; 1.4148x over previous
import functools

import jax
import jax.numpy as jnp
from jax.experimental import pallas as pl
from jax.experimental.pallas import tpu as pltpu

TOPK = 8
NE = 64
NEG_INF = float("-inf")


def _router_block(x_ref, w_ref, b_ref, eb_ref, gw_ref, idx_ref):
    x = x_ref[...]
    w = w_ref[...]
    scores = jax.lax.dot_general(
        x, w, (((1,), (0,)), ((), ())),
        preferred_element_type=jnp.float32,
    )
    scores = scores + b_ref[...]
    sel = scores + eb_ref[...]
    iota = jax.lax.broadcasted_iota(jnp.int32, sel.shape, 1)
    cur = sel
    vals = []
    idxs = []
    for _ in range(TOPK):
        m = jnp.max(cur, axis=1, keepdims=True)
        idx = jnp.min(jnp.where(cur == m, iota, NE), axis=1, keepdims=True)
        chosen = iota == idx
        vals.append(jnp.sum(jnp.where(chosen, scores, 0.0), axis=1,
                            keepdims=True))
        idxs.append(idx)
        cur = jnp.where(chosen, NEG_INF, cur)
    g = jnp.concatenate(vals, axis=1)
    m8 = jnp.max(g, axis=1, keepdims=True)
    e8 = jnp.exp(g - m8)
    gw_ref[...] = e8 / jnp.sum(e8, axis=1, keepdims=True)
    idx_ref[...] = jnp.concatenate(idxs, axis=1)


@jax.jit
def kernel(x_router_input, W, b, expert_biases):
    n_tokens, d_model = x_router_input.shape
    T = 512
    grid = (n_tokens // T,)
    b2 = b.reshape(1, NE)
    eb2 = expert_biases.reshape(1, NE)
    gw, idx = pl.pallas_call(
        _router_block,
        grid=grid,
        in_specs=[
            pl.BlockSpec((T, d_model), lambda i: (i, 0)),
            pl.BlockSpec((d_model, NE), lambda i: (0, 0)),
            pl.BlockSpec((1, NE), lambda i: (0, 0)),
            pl.BlockSpec((1, NE), lambda i: (0, 0)),
        ],
        out_specs=[
            pl.BlockSpec((T, TOPK), lambda i: (i, 0)),
            pl.BlockSpec((T, TOPK), lambda i: (i, 0)),
        ],
        out_shape=[
            jax.ShapeDtypeStruct((n_tokens, TOPK), jnp.float32),
            jax.ShapeDtypeStruct((n_tokens, TOPK), jnp.int32),
        ],
    )(x_router_input, W, b2, eb2)
    return gw, idx
